# Initial kernel scaffold; baseline (speedup 1.0000x reference)
import jax
import jax.numpy as jnp
from jax.experimental import pallas as pl


def kernel(x, Wg, Wu, Wd, g_Wr, g_A, g_B, u_Wr, u_A, u_B, d_Wr, d_A, d_B):
    raise NotImplementedError("write your pallas kernel here")


# speedup vs baseline: 6.8329x; 6.8329x over previous
import functools

import jax
import jax.numpy as jnp
from jax.experimental import pallas as pl

H = 1024
I = 4096
E = 8
R = 8
SCALE = 16.0 / 8.0
TN = 256


def _expand_gates(gates):
    eidx = jax.lax.broadcasted_iota(jnp.int32, (E, E * R), 0)
    cidx = jax.lax.broadcasted_iota(jnp.int32, (E, E * R), 1)
    sel = jnp.where(cidx // R == eidx, SCALE, 0.0).astype(jnp.float32)
    return jnp.dot(gates, sel, preferred_element_type=jnp.float32)


def _tmoe_delta(xb, wr, ac, bc):
    logits = jnp.dot(xb, wr, preferred_element_type=jnp.float32)
    m = jnp.max(logits, axis=-1, keepdims=True)
    ex = jnp.exp(logits - m)
    gates = ex / jnp.sum(ex, axis=-1, keepdims=True)
    h = jnp.dot(xb, ac, preferred_element_type=jnp.float32)
    wh = (h * _expand_gates(gates)).astype(jnp.bfloat16)
    return jnp.dot(wh, bc, preferred_element_type=jnp.float32)


def _fused_kernel(x_ref, wg_ref, wu_ref, wd_ref,
                  gwr_ref, gac_ref, gbc_ref,
                  uwr_ref, uac_ref, ubc_ref,
                  dwr_ref, dac_ref, dbc_ref,
                  out_ref):
    xb = x_ref[...]
    gate = jnp.dot(xb, wg_ref[...], preferred_element_type=jnp.float32)
    gate += _tmoe_delta(xb, gwr_ref[...], gac_ref[...], gbc_ref[...])
    up = jnp.dot(xb, wu_ref[...], preferred_element_type=jnp.float32)
    up += _tmoe_delta(xb, uwr_ref[...], uac_ref[...], ubc_ref[...])
    inter = (gate * jax.lax.logistic(gate) * up).astype(jnp.bfloat16)
    out = jnp.dot(inter, wd_ref[...], preferred_element_type=jnp.float32)
    out += _tmoe_delta(inter, dwr_ref[...], dac_ref[...], dbc_ref[...])
    out_ref[...] = out


@functools.partial(jax.jit, static_argnames=())
def kernel(x, Wg, Wu, Wd, g_Wr, g_A, g_B, u_Wr, u_A, u_B, d_Wr, d_A, d_B):
    Bb, Ss, Hh = x.shape
    N = Bb * Ss
    xb = x.reshape(N, Hh).astype(jnp.bfloat16)

    def prep(Wr, A, Bm):
        ac = A.transpose(1, 0, 2).reshape(A.shape[1], E * R).astype(jnp.bfloat16)
        bc = Bm.reshape(E * R, Bm.shape[2]).astype(jnp.bfloat16)
        return Wr.astype(jnp.bfloat16), ac, bc

    gwr, gac, gbc = prep(g_Wr, g_A, g_B)
    uwr, uac, ubc = prep(u_Wr, u_A, u_B)
    dwr, dac, dbc = prep(d_Wr, d_A, d_B)
    wg = Wg.astype(jnp.bfloat16)
    wu = Wu.astype(jnp.bfloat16)
    wd = Wd.astype(jnp.bfloat16)

    full = lambda shape: pl.BlockSpec(shape, lambda i: (0, 0))
    out = pl.pallas_call(
        _fused_kernel,
        grid=(N // TN,),
        in_specs=[
            pl.BlockSpec((TN, H), lambda i: (i, 0)),
            full((H, I)), full((H, I)), full((I, H)),
            full((H, E)), full((H, E * R)), full((E * R, I)),
            full((H, E)), full((H, E * R)), full((E * R, I)),
            full((I, E)), full((I, E * R)), full((E * R, H)),
        ],
        out_specs=pl.BlockSpec((TN, H), lambda i: (i, 0)),
        out_shape=jax.ShapeDtypeStruct((N, H), jnp.float32),
    )(xb, wg, wu, wd, gwr, gac, gbc, uwr, uac, ubc, dwr, dac, dbc)
    return out.reshape(Bb, Ss, Hh)


# speedup vs baseline: 7.2330x; 1.0586x over previous
import functools

import jax
import jax.numpy as jnp
from jax.experimental import pallas as pl

H = 1024
I = 4096
E = 8
R = 8
SCALE = 16.0 / 8.0
TN = 512


def _expand_gates(gates):
    eidx = jax.lax.broadcasted_iota(jnp.int32, (E, E * R), 0)
    cidx = jax.lax.broadcasted_iota(jnp.int32, (E, E * R), 1)
    sel = jnp.where(cidx // R == eidx, SCALE, 0.0).astype(jnp.float32)
    return jnp.dot(gates, sel, preferred_element_type=jnp.float32)


def _tmoe_delta(xb, wr, ac, bc):
    logits = jnp.dot(xb, wr, preferred_element_type=jnp.float32)
    m = jnp.max(logits, axis=-1, keepdims=True)
    ex = jnp.exp(logits - m)
    gates = ex / jnp.sum(ex, axis=-1, keepdims=True)
    h = jnp.dot(xb, ac, preferred_element_type=jnp.float32)
    wh = (h * _expand_gates(gates)).astype(jnp.bfloat16)
    return jnp.dot(wh, bc, preferred_element_type=jnp.float32)


def _fused_kernel(x_ref, wg_ref, wu_ref, wd_ref,
                  gwr_ref, gac_ref, gbc_ref,
                  uwr_ref, uac_ref, ubc_ref,
                  dwr_ref, dac_ref, dbc_ref,
                  out_ref):
    xb = x_ref[...].astype(jnp.bfloat16)
    gate = jnp.dot(xb, wg_ref[...], preferred_element_type=jnp.float32)
    gate += _tmoe_delta(xb, gwr_ref[...], gac_ref[...], gbc_ref[...])
    up = jnp.dot(xb, wu_ref[...], preferred_element_type=jnp.float32)
    up += _tmoe_delta(xb, uwr_ref[...], uac_ref[...], ubc_ref[...])
    inter = (gate * jax.lax.logistic(gate) * up).astype(jnp.bfloat16)
    out = jnp.dot(inter, wd_ref[...], preferred_element_type=jnp.float32)
    out += _tmoe_delta(inter, dwr_ref[...], dac_ref[...], dbc_ref[...])
    out_ref[...] = out


@functools.partial(jax.jit, static_argnames=())
def kernel(x, Wg, Wu, Wd, g_Wr, g_A, g_B, u_Wr, u_A, u_B, d_Wr, d_A, d_B):
    Bb, Ss, Hh = x.shape
    N = Bb * Ss
    xb = x.reshape(N, Hh)

    def prep(Wr, A, Bm):
        ac = A.transpose(1, 0, 2).reshape(A.shape[1], E * R).astype(jnp.bfloat16)
        bc = Bm.reshape(E * R, Bm.shape[2]).astype(jnp.bfloat16)
        return Wr.astype(jnp.bfloat16), ac, bc

    gwr, gac, gbc = prep(g_Wr, g_A, g_B)
    uwr, uac, ubc = prep(u_Wr, u_A, u_B)
    dwr, dac, dbc = prep(d_Wr, d_A, d_B)
    wg = Wg.astype(jnp.bfloat16)
    wu = Wu.astype(jnp.bfloat16)
    wd = Wd.astype(jnp.bfloat16)

    full = lambda shape: pl.BlockSpec(shape, lambda i: (0, 0))
    out = pl.pallas_call(
        _fused_kernel,
        grid=(N // TN,),
        in_specs=[
            pl.BlockSpec((TN, H), lambda i: (i, 0)),
            full((H, I)), full((H, I)), full((I, H)),
            full((H, E)), full((H, E * R)), full((E * R, I)),
            full((H, E)), full((H, E * R)), full((E * R, I)),
            full((I, E)), full((I, E * R)), full((E * R, H)),
        ],
        out_specs=pl.BlockSpec((TN, H), lambda i: (i, 0)),
        out_shape=jax.ShapeDtypeStruct((N, H), jnp.float32),
    )(xb, wg, wu, wd, gwr, gac, gbc, uwr, uac, ubc, dwr, dac, dbc)
    return out.reshape(Bb, Ss, Hh)


# speedup vs baseline: 7.2862x; 1.0074x over previous
import functools

import jax
import jax.numpy as jnp
from jax.experimental import pallas as pl

H = 1024
I = 4096
E = 8
R = 8
ER = E * R
SCALE = 16.0 / 8.0
TN = 512
CK = 1024
NCK = I // CK


def _expand_gates(logits):
    m = jnp.max(logits, axis=-1, keepdims=True)
    ex = jnp.exp(logits - m)
    gates = ex / jnp.sum(ex, axis=-1, keepdims=True)
    eidx = jax.lax.broadcasted_iota(jnp.int32, (E, ER), 0)
    cidx = jax.lax.broadcasted_iota(jnp.int32, (E, ER), 1)
    sel = jnp.where(cidx // R == eidx, SCALE, 0.0).astype(jnp.float32)
    return jnp.dot(gates, sel, preferred_element_type=jnp.float32)


def _fused_kernel(x_ref, wg_ref, wu_ref, wd_ref, guc_ref, dc_ref,
                  gbc_ref, ubc_ref, dbc_ref, out_ref):
    xb = x_ref[...].astype(jnp.bfloat16)

    r = jnp.dot(xb, guc_ref[...], preferred_element_type=jnp.float32)
    whg = (r[:, 16:80] * _expand_gates(r[:, 0:8])).astype(jnp.bfloat16)
    whu = (r[:, 80:144] * _expand_gates(r[:, 8:16])).astype(jnp.bfloat16)

    out_acc = jnp.zeros((TN, H), jnp.float32)
    dr_acc = jnp.zeros((TN, E + ER), jnp.float32)
    for k in range(NCK):
        sl = pl.ds(k * CK, CK)
        g_k = jnp.dot(xb, wg_ref[:, sl], preferred_element_type=jnp.float32)
        g_k += jnp.dot(whg, gbc_ref[:, sl], preferred_element_type=jnp.float32)
        u_k = jnp.dot(xb, wu_ref[:, sl], preferred_element_type=jnp.float32)
        u_k += jnp.dot(whu, ubc_ref[:, sl], preferred_element_type=jnp.float32)
        inter_k = (g_k * jax.lax.logistic(g_k) * u_k).astype(jnp.bfloat16)
        out_acc += jnp.dot(inter_k, wd_ref[sl, :],
                           preferred_element_type=jnp.float32)
        dr_acc += jnp.dot(inter_k, dc_ref[sl, :],
                          preferred_element_type=jnp.float32)

    whd = (dr_acc[:, E:] * _expand_gates(dr_acc[:, :E])).astype(jnp.bfloat16)
    out_ref[...] = out_acc + jnp.dot(whd, dbc_ref[...],
                                     preferred_element_type=jnp.float32)


@functools.partial(jax.jit, static_argnames=())
def kernel(x, Wg, Wu, Wd, g_Wr, g_A, g_B, u_Wr, u_A, u_B, d_Wr, d_A, d_B):
    Bb, Ss, Hh = x.shape
    N = Bb * Ss
    xf = x.reshape(N, Hh)

    def acat(A):
        return A.transpose(1, 0, 2).reshape(A.shape[1], ER)

    guc = jnp.concatenate([g_Wr, u_Wr, acat(g_A), acat(u_A)],
                          axis=1).astype(jnp.bfloat16)
    dc = jnp.concatenate([d_Wr, acat(d_A)], axis=1).astype(jnp.bfloat16)
    gbc = g_B.reshape(ER, I).astype(jnp.bfloat16)
    ubc = u_B.reshape(ER, I).astype(jnp.bfloat16)
    dbc = d_B.reshape(ER, H).astype(jnp.bfloat16)
    wg = Wg.astype(jnp.bfloat16)
    wu = Wu.astype(jnp.bfloat16)
    wd = Wd.astype(jnp.bfloat16)

    full = lambda shape: pl.BlockSpec(shape, lambda i: (0, 0))
    out = pl.pallas_call(
        _fused_kernel,
        grid=(N // TN,),
        in_specs=[
            pl.BlockSpec((TN, H), lambda i: (i, 0)),
            full((H, I)), full((H, I)), full((I, H)),
            full((H, 2 * E + 2 * ER)), full((I, E + ER)),
            full((ER, I)), full((ER, I)), full((ER, H)),
        ],
        out_specs=pl.BlockSpec((TN, H), lambda i: (i, 0)),
        out_shape=jax.ShapeDtypeStruct((N, H), jnp.float32),
    )(xf, wg, wu, wd, guc, dc, gbc, ubc, dbc)
    return out.reshape(Bb, Ss, Hh)


# speedup vs baseline: 7.4500x; 1.0225x over previous
import functools

import jax
import jax.numpy as jnp
from jax.experimental import pallas as pl

H = 1024
I = 4096
E = 8
R = 8
ER = E * R
SCALE = 16.0 / 8.0
TN = 1024
CK = 1024
NCK = I // CK


def _expand_gates(logits):
    m = jnp.max(logits, axis=-1, keepdims=True)
    ex = jnp.exp(logits - m)
    gates = ex / jnp.sum(ex, axis=-1, keepdims=True)
    eidx = jax.lax.broadcasted_iota(jnp.int32, (E, ER), 0)
    cidx = jax.lax.broadcasted_iota(jnp.int32, (E, ER), 1)
    sel = jnp.where(cidx // R == eidx, SCALE, 0.0).astype(jnp.float32)
    return jnp.dot(gates, sel, preferred_element_type=jnp.float32)


def _fused_kernel(x_ref, wg_ref, wu_ref, wd_ref, guc_ref, dc_ref,
                  gbc_ref, ubc_ref, dbc_ref, out_ref):
    xb = x_ref[...].astype(jnp.bfloat16)

    r = jnp.dot(xb, guc_ref[...], preferred_element_type=jnp.float32)
    whg = (r[:, 16:80] * _expand_gates(r[:, 0:8])).astype(jnp.bfloat16)
    whu = (r[:, 80:144] * _expand_gates(r[:, 8:16])).astype(jnp.bfloat16)

    out_acc = jnp.zeros((TN, H), jnp.float32)
    dr_acc = jnp.zeros((TN, E + ER), jnp.float32)
    for k in range(NCK):
        sl = pl.ds(k * CK, CK)
        g_k = jnp.dot(xb, wg_ref[:, sl], preferred_element_type=jnp.float32)
        g_k += jnp.dot(whg, gbc_ref[:, sl], preferred_element_type=jnp.float32)
        u_k = jnp.dot(xb, wu_ref[:, sl], preferred_element_type=jnp.float32)
        u_k += jnp.dot(whu, ubc_ref[:, sl], preferred_element_type=jnp.float32)
        inter_k = (g_k * jax.lax.logistic(g_k) * u_k).astype(jnp.bfloat16)
        out_acc += jnp.dot(inter_k, wd_ref[sl, :],
                           preferred_element_type=jnp.float32)
        dr_acc += jnp.dot(inter_k, dc_ref[sl, :],
                          preferred_element_type=jnp.float32)

    whd = (dr_acc[:, E:] * _expand_gates(dr_acc[:, :E])).astype(jnp.bfloat16)
    out_ref[...] = out_acc + jnp.dot(whd, dbc_ref[...],
                                     preferred_element_type=jnp.float32)


@functools.partial(jax.jit, static_argnames=())
def kernel(x, Wg, Wu, Wd, g_Wr, g_A, g_B, u_Wr, u_A, u_B, d_Wr, d_A, d_B):
    Bb, Ss, Hh = x.shape
    N = Bb * Ss
    xf = x.reshape(N, Hh)

    def acat(A):
        return A.transpose(1, 0, 2).reshape(A.shape[1], ER)

    guc = jnp.concatenate([g_Wr, u_Wr, acat(g_A), acat(u_A)],
                          axis=1).astype(jnp.bfloat16)
    dc = jnp.concatenate([d_Wr, acat(d_A)], axis=1).astype(jnp.bfloat16)
    gbc = g_B.reshape(ER, I).astype(jnp.bfloat16)
    ubc = u_B.reshape(ER, I).astype(jnp.bfloat16)
    dbc = d_B.reshape(ER, H).astype(jnp.bfloat16)
    wg = Wg.astype(jnp.bfloat16)
    wu = Wu.astype(jnp.bfloat16)
    wd = Wd.astype(jnp.bfloat16)

    full = lambda shape: pl.BlockSpec(shape, lambda i: (0, 0))
    out = pl.pallas_call(
        _fused_kernel,
        grid=(N // TN,),
        in_specs=[
            pl.BlockSpec((TN, H), lambda i: (i, 0)),
            full((H, I)), full((H, I)), full((I, H)),
            full((H, 2 * E + 2 * ER)), full((I, E + ER)),
            full((ER, I)), full((ER, I)), full((ER, H)),
        ],
        out_specs=pl.BlockSpec((TN, H), lambda i: (i, 0)),
        out_shape=jax.ShapeDtypeStruct((N, H), jnp.float32),
    )(xf, wg, wu, wd, guc, dc, gbc, ubc, dbc)
    return out.reshape(Bb, Ss, Hh)


# speedup vs baseline: 7.5625x; 1.0151x over previous
import functools

import jax
import jax.numpy as jnp
from jax.experimental import pallas as pl
from jax.experimental.pallas import tpu as pltpu

H = 1024
I = 4096
E = 8
R = 8
ER = E * R
SCALE = 16.0 / 8.0
TN = 512
CK = 1024
NCK = I // CK
RCH = 128


def _expand_gates(logits):
    m = jnp.max(logits, axis=-1, keepdims=True)
    ex = jnp.exp(logits - m)
    gates = ex / jnp.sum(ex, axis=-1, keepdims=True)
    eidx = jax.lax.broadcasted_iota(jnp.int32, (E, ER), 0)
    cidx = jax.lax.broadcasted_iota(jnp.int32, (E, ER), 1)
    sel = jnp.where(cidx // R == eidx, SCALE, 0.0).astype(jnp.float32)
    return jnp.dot(gates, sel, preferred_element_type=jnp.float32)


def _stage_cast(src_hbm, dst_vmem, st, sem, nrows, rch):
    nch = nrows // rch

    def cp(c, slot):
        return pltpu.make_async_copy(
            src_hbm.at[pl.ds(c * rch, rch), :], st.at[slot], sem.at[slot])

    cp(0, 0).start()

    def body(c, _):
        slot = jax.lax.rem(c, 2)
        nslot = jax.lax.rem(c + 1, 2)

        @pl.when(c + 1 < nch)
        def _():
            cp(c + 1, nslot).start()

        cp(c, slot).wait()
        dst_vmem[pl.ds(c * rch, rch), :] = st[slot].astype(jnp.bfloat16)
        return 0

    jax.lax.fori_loop(0, nch, body, 0)


def _fused_kernel(x_ref, wg_hbm, wu_hbm, wd_hbm, guc_ref, dc_ref,
                  gbc_ref, ubc_ref, dbc_ref, out_ref,
                  wg_ref, wu_ref, wd_ref, st_ref, std_ref, sem):
    @pl.when(pl.program_id(0) == 0)
    def _prologue():
        _stage_cast(wg_hbm, wg_ref, st_ref, sem, H, RCH)
        _stage_cast(wu_hbm, wu_ref, st_ref, sem, H, RCH)
        _stage_cast(wd_hbm, wd_ref, std_ref, sem, I, 4 * RCH)

    xb = x_ref[...].astype(jnp.bfloat16)

    r = jnp.dot(xb, guc_ref[...], preferred_element_type=jnp.float32)
    whg = (r[:, 16:80] * _expand_gates(r[:, 0:8])).astype(jnp.bfloat16)
    whu = (r[:, 80:144] * _expand_gates(r[:, 8:16])).astype(jnp.bfloat16)

    out_acc = jnp.zeros((TN, H), jnp.float32)
    dr_acc = jnp.zeros((TN, E + ER), jnp.float32)
    for k in range(NCK):
        sl = pl.ds(k * CK, CK)
        g_k = jnp.dot(xb, wg_ref[:, sl], preferred_element_type=jnp.float32)
        g_k += jnp.dot(whg, gbc_ref[:, sl], preferred_element_type=jnp.float32)
        u_k = jnp.dot(xb, wu_ref[:, sl], preferred_element_type=jnp.float32)
        u_k += jnp.dot(whu, ubc_ref[:, sl], preferred_element_type=jnp.float32)
        inter_k = (g_k * jax.lax.logistic(g_k) * u_k).astype(jnp.bfloat16)
        out_acc += jnp.dot(inter_k, wd_ref[sl, :],
                           preferred_element_type=jnp.float32)
        dr_acc += jnp.dot(inter_k, dc_ref[sl, :],
                          preferred_element_type=jnp.float32)

    whd = (dr_acc[:, E:] * _expand_gates(dr_acc[:, :E])).astype(jnp.bfloat16)
    out_ref[...] = out_acc + jnp.dot(whd, dbc_ref[...],
                                     preferred_element_type=jnp.float32)


@functools.partial(jax.jit, static_argnames=())
def kernel(x, Wg, Wu, Wd, g_Wr, g_A, g_B, u_Wr, u_A, u_B, d_Wr, d_A, d_B):
    Bb, Ss, Hh = x.shape
    N = Bb * Ss
    xf = x.reshape(N, Hh)

    def acat(A):
        return A.transpose(1, 0, 2).reshape(A.shape[1], ER)

    guc = jnp.concatenate([g_Wr, u_Wr, acat(g_A), acat(u_A)],
                          axis=1).astype(jnp.bfloat16)
    dc = jnp.concatenate([d_Wr, acat(d_A)], axis=1).astype(jnp.bfloat16)
    gbc = g_B.reshape(ER, I).astype(jnp.bfloat16)
    ubc = u_B.reshape(ER, I).astype(jnp.bfloat16)
    dbc = d_B.reshape(ER, H).astype(jnp.bfloat16)

    full = lambda shape: pl.BlockSpec(shape, lambda i: (0, 0))
    hbm = pl.BlockSpec(memory_space=pl.ANY)
    out = pl.pallas_call(
        _fused_kernel,
        grid=(N // TN,),
        in_specs=[
            pl.BlockSpec((TN, H), lambda i: (i, 0)),
            hbm, hbm, hbm,
            full((H, 2 * E + 2 * ER)), full((I, E + ER)),
            full((ER, I)), full((ER, I)), full((ER, H)),
        ],
        out_specs=pl.BlockSpec((TN, H), lambda i: (i, 0)),
        out_shape=jax.ShapeDtypeStruct((N, H), jnp.float32),
        scratch_shapes=[
            pltpu.VMEM((H, I), jnp.bfloat16),
            pltpu.VMEM((H, I), jnp.bfloat16),
            pltpu.VMEM((I, H), jnp.bfloat16),
            pltpu.VMEM((2, RCH, I), jnp.float32),
            pltpu.VMEM((2, 4 * RCH, H), jnp.float32),
            pltpu.SemaphoreType.DMA((2,)),
        ],
    )(xf, Wg, Wu, Wd, guc, dc, gbc, ubc, dbc)
    return out.reshape(Bb, Ss, Hh)
